# baseline (speedup 1.0000x reference)
import jax
import jax.numpy as jnp
from jax.experimental import pallas as pl


def VSC(inputs):
    import functools
    from jax import lax
    from jax.experimental.pallas import tpu as pltpu
    from jax.experimental.pallas import tpu_sc as plsc

    b, d, ch = inputs.shape
    x_t = inputs.transpose(0, 2, 1)
    mesh = plsc.VectorSubcoreMesh(core_axis_name="c", subcore_axis_name="s")

    @functools.partial(
        pl.kernel,
        mesh=mesh,
        out_type=jax.ShapeDtypeStruct((9, b, 1024), jnp.float32),
        scratch_types=[
            pltpu.VMEM((32, 1024), jnp.float32),
            pltpu.VMEM((64, 1024), jnp.float32),
        ],
    )
    def _sc_ed(x_hbm, nan_hbm, ed_hbm, nanbuf, rowbuf):
        wid = lax.axis_index("s") * 2 + lax.axis_index("c")
        pltpu.sync_copy(nan_hbm, nanbuf)
        p = wid // 4
        b0 = (wid % 4) * 1024

        @pl.loop(0, 32)
        def _(i):
            pltpu.sync_copy(nanbuf, ed_hbm.at[p, pl.ds(b0 + i * 32, 32)])

        @pl.loop(0, 2)
        def _(h):
            r0 = wid * 128 + h * 64

            @pl.loop(0, 64)
            def _(j):
                pltpu.sync_copy(x_hbm.at[r0 + j, 7], rowbuf.at[j])

            pltpu.sync_copy(rowbuf, ed_hbm.at[8, pl.ds(r0, 64)])

    nanrows = jnp.full((32, 1024), jnp.nan, jnp.float32)
    ed_t = _sc_ed(x_t, nanrows)

    bb = 256

    def _pd_body(x_ref, pd_ref):
        x = x_ref[...]
        s = jax.lax.broadcasted_iota(jnp.int32, x.shape, 1)
        pd_ref[...] = jnp.where(s == 7, jnp.nan, x)

    pd_t = pl.pallas_call(
        _pd_body,
        grid=(b // bb,),
        in_specs=[pl.BlockSpec((bb, 8, 1024), lambda i: (i, 0, 0))],
        out_specs=pl.BlockSpec((bb, 8, 1024), lambda i: (i, 0, 0)),
        out_shape=jax.ShapeDtypeStruct((b, 8, 1024), jnp.float32),
    )(x_t)
    return pd_t.transpose(0, 2, 1), ed_t.transpose(1, 2, 0)


@jax.jit
def kernel(inputs):
    return VSC(inputs)


# speedup vs baseline: 1.7394x; 1.7394x over previous
import jax
import jax.numpy as jnp
from jax.experimental import pallas as pl
from jax.experimental.pallas import tpu as pltpu

_B = 1024


def _split_body(x_ref, pd_ref, ed_ref, c7_ref):
    c = pl.program_id(1)
    b8 = x_ref.shape[0]

    @pl.when(c < 8)
    def _():
        x = x_ref[...]
        s = jax.lax.broadcasted_iota(jnp.int32, x.shape, 1)
        pd_ref[...] = jnp.where(s == 7, jnp.nan, x)
        c7_ref[pl.ds(c * b8, b8), :] = x[:, 7, :]
        ed_ref[...] = jnp.full(ed_ref.shape, jnp.nan, jnp.float32)

    @pl.when(c == 8)
    def _():
        ed_ref[0] = c7_ref[...]


@jax.jit
def kernel(inputs):
    b, d, ch = inputs.shape
    b8 = _B // 8
    x_t = inputs.transpose(0, 2, 1)
    pd_t, ed_t = pl.pallas_call(
        _split_body,
        grid=(b // _B, 9),
        in_specs=[
            pl.BlockSpec((b8, 8, 1024), lambda i, c: (8 * i + jnp.minimum(c, 7), 0, 0))
        ],
        out_specs=[
            pl.BlockSpec((b8, 8, 1024), lambda i, c: (8 * i + jnp.minimum(c, 7), 0, 0)),
            pl.BlockSpec((1, _B, 1024), lambda i, c: (c, i, 0)),
        ],
        out_shape=[
            jax.ShapeDtypeStruct((b, 8, 1024), jnp.float32),
            jax.ShapeDtypeStruct((9, b, 1024), jnp.float32),
        ],
        scratch_shapes=[pltpu.VMEM((_B, 1024), jnp.float32)],
    )(x_t)
    return pd_t.transpose(0, 2, 1), ed_t.transpose(1, 2, 0)
